# baseline (speedup 1.0000x reference)
import functools
import math

import jax
import jax.numpy as jnp
from jax import lax
from jax.experimental import pallas as pl
from jax.experimental.pallas import tpu as pltpu
from jax.experimental.pallas import tpu_sc as plsc

_NC = 2
_NS = 16
_NW = _NC * _NS
_CHUNK = 128
_BN_C = 1.0 / math.sqrt(1.0 + 1e-5)


def _row_init(sid, src, dst, rows_pt, rows_tail):
  base_r = sid * rows_pt
  pltpu.sync_copy(src.at[pl.ds(base_r, rows_pt)], dst.at[pl.ds(base_r, rows_pt)])
  if rows_tail:
    @pl.when(sid == _NS - 1)
    def _tail():
      pltpu.sync_copy(src.at[pl.ds(rows_pt * _NS, rows_tail)],
                      dst.at[pl.ds(rows_pt * _NS, rows_tail)])


_PADR = 16


def _splits(n, d, e):
  epw = e // _NW
  nchunks = -(-epw // _CHUNK)
  assert epw * _NW == e and d % 128 == 0
  rows_pt = (n // _NS) & ~7
  rows_tail = n - rows_pt * _NS
  assert rows_tail >= 0 and rows_tail % 8 == 0
  return nchunks, rows_pt, rows_tail


@functools.lru_cache(maxsize=None)
def _make_agg(n, d, e):
  nchunks, rows_pt, rows_tail = _splits(n, d, e)
  nmain = (nchunks // 4) * 4
  tail = list(range(nmain, nchunks))
  mesh = plsc.VectorSubcoreMesh(core_axis_name="c", subcore_axis_name="s")

  @functools.partial(
      pl.kernel,
      out_type=jax.ShapeDtypeStruct((_NC, n, d), jnp.float32),
      mesh=mesh,
      scratch_types=[
          pltpu.VMEM((8, _CHUNK), jnp.int32),
          pltpu.VMEM((2, _CHUNK, d), jnp.float32),
          pltpu.VMEM_SHARED((n + _PADR, d), jnp.float32),
      ] + [pltpu.SemaphoreType.DMA] * 8,
  )
  def agg(u_hbm, rc_hbm, out_hbm, rcring, gbuf, acc, *sems):
    gsem = sems[0:2]
    ssem = sems[2:4]
    isem = sems[4:8]
    cid = lax.axis_index("c")
    sid = lax.axis_index("s")
    wid = sid * _NC + cid
    _row_init(sid, u_hbm, acc, rows_pt, rows_tail)

    def start_fetch(ci, s4):
      pltpu.async_copy(rc_hbm.at[wid].at[pl.ds(2 * ci, 2)],
                       rcring.at[pl.ds(2 * s4, 2)], isem[s4])

    def wait_fetch(s4):
      pltpu.make_async_copy(rc_hbm.at[0].at[pl.ds(0, 2)],
                            rcring.at[pl.ds(2 * s4, 2)], isem[s4]).wait()

    def start_gather(s4, g):
      pltpu.async_copy(u_hbm.at[rcring.at[2 * s4]], gbuf.at[g], gsem[g])

    def wait_gather(g):
      pltpu.make_async_copy(u_hbm.at[rcring.at[0]], gbuf.at[g],
                            gsem[g]).wait()

    def start_scatter(s4, g):
      pltpu.async_copy(gbuf.at[g], acc.at[rcring.at[2 * s4 + 1]], ssem[g],
                       add=True)

    def wait_scatter(g):
      pltpu.make_async_copy(gbuf.at[g], acc.at[rcring.at[0]], ssem[g]).wait()

    for c in range(2):
      start_fetch(c, c)
    plsc.subcore_barrier()

    def step(ci, b):
      if isinstance(ci, int):
        if ci >= 2:
          wait_scatter(b % 2)
        if ci + 2 < nchunks:
          start_fetch(ci + 2, (b + 2) % 4)
      else:
        @pl.when(ci >= 2)
        def _recycle():
          wait_scatter(b % 2)

        @pl.when(ci + 2 < nchunks)
        def _pref():
          start_fetch(ci + 2, (b + 2) % 4)

      wait_fetch(b)
      start_gather(b, b % 2)
      wait_gather(b % 2)
      start_scatter(b, b % 2)

    def body(k, carry):
      for b in range(4):
        step(4 * k + b, b)
      return carry

    lax.fori_loop(0, nmain // 4, body, 0)
    for ci in tail:
      step(ci, ci % 4)
    for ci in (nchunks - 2, nchunks - 1):
      wait_scatter(ci % 2)
    plsc.subcore_barrier()
    _row_init(sid, acc, out_hbm.at[cid], rows_pt, rows_tail)

  return agg


_DEGW = 64


@functools.lru_cache(maxsize=None)
def _make_deg(n, d, e):
  nchunks, rows_pt, rows_tail = _splits(n, d, e)
  window = 8
  mesh = plsc.VectorSubcoreMesh(core_axis_name="c", subcore_axis_name="s")

  @functools.partial(
      pl.kernel,
      out_type=jax.ShapeDtypeStruct((_NC, n, _DEGW), jnp.float32),
      mesh=mesh,
      scratch_types=[
          pltpu.VMEM((nchunks, _CHUNK), jnp.int32),
          pltpu.VMEM((_CHUNK, _DEGW), jnp.float32),
          pltpu.VMEM_SHARED((n + _PADR, _DEGW), jnp.float32),
          pltpu.SemaphoreType.DMA,
      ],
  )
  def deg(half_hbm, ones_hbm, col3_hbm, out_hbm, col_v, ones_v, acc, ssem):
    cid = lax.axis_index("c")
    sid = lax.axis_index("s")
    wid = sid * _NC + cid
    pltpu.sync_copy(col3_hbm.at[wid], col_v)
    pltpu.sync_copy(ones_hbm.at[pl.ds(0, _CHUNK)], ones_v)
    _row_init(sid, half_hbm, acc, rows_pt, rows_tail)
    plsc.subcore_barrier()

    def start_sc(ci):
      pltpu.async_copy(ones_v, acc.at[col_v.at[ci]], ssem, add=True)

    def wait_sc(i, carry):
      pltpu.make_async_copy(ones_v, acc.at[col_v.at[0]], ssem).wait()
      return carry

    for ci in range(window):
      start_sc(ci)

    def body(k, carry):
      wait_sc(k, carry)
      start_sc(k + window)
      return carry

    lax.fori_loop(0, nchunks - window, body, 0)
    lax.fori_loop(0, window, wait_sc, 0)
    plsc.subcore_barrier()
    _row_init(sid, acc, out_hbm.at[cid], rows_pt, rows_tail)

  return deg


def _first_body(x, d0, d1, w, o_u, o_dis):
  deg = d0[...][:, :1] + d1[...][:, :1]
  dis = jnp.broadcast_to(lax.rsqrt(deg), o_dis.shape)
  o_dis[...] = dis
  o_u[...] = jnp.dot(x[...], w[...], preferred_element_type=jnp.float32) * dis


def _mid_body(a0, a1, up, dis, b, g, be, w, o):
  d = dis[...]
  z = (a0[...] + a1[...] - up[...]) * d + b[...]
  z = z * (g[...] * _BN_C) + be[...]
  z = jnp.maximum(z, 0.0)
  o[...] = jnp.dot(z, w[...], preferred_element_type=jnp.float32) * d


def _last_body(a0, a1, up, dis, b, g, be, o):
  z = (a0[...] + a1[...] - up[...]) * dis[...] + b[...]
  o[...] = z * (g[...] * _BN_C) + be[...]


def _tc_call(body, n, d, r, arrs, vecs, weights, num_out=1, num_narrow=0):
  grid = (n // r,)
  nd_spec = pl.BlockSpec((r, d), lambda i: (i, 0))
  nr_spec = pl.BlockSpec((r, _DEGW), lambda i: (i, 0))
  vec_spec = pl.BlockSpec((1, d), lambda i: (0, 0))
  w_spec = pl.BlockSpec((d, d), lambda i: (0, 0))
  in_specs = ([nd_spec] * (len(arrs) - num_narrow) + [nr_spec] * num_narrow
              + [vec_spec] * len(vecs) + [w_spec] * len(weights))
  shape = jax.ShapeDtypeStruct((n, d), jnp.float32)
  return pl.pallas_call(
      body,
      grid=grid,
      in_specs=in_specs,
      out_specs=[nd_spec] * num_out if num_out > 1 else nd_spec,
      out_shape=[shape] * num_out if num_out > 1 else shape,
  )(*arrs, *vecs, *weights)


def kernel(x, edge_index, W1, b1, g1, be1, W2, b2, g2, be2, W3, b3, g3, be3):
  n, d = x.shape
  e = edge_index.shape[1]
  epw = e // _NW
  nchunks = -(-epw // _CHUNK)
  pad = nchunks * _CHUNK - epw

  w_ids = jnp.arange(_NW, dtype=jnp.int32)[:, None]
  p_ids = jnp.arange(pad, dtype=jnp.int32)[None, :]
  rowp = jnp.concatenate(
      [edge_index[0].reshape(_NW, epw), (p_ids + 113 * w_ids) % n], axis=1)
  colp = jnp.concatenate(
      [edge_index[1].reshape(_NW, epw), n + (p_ids + w_ids) % _PADR], axis=1)
  col3 = colp.reshape(_NW, nchunks, _CHUNK)
  rc = jnp.stack([rowp.reshape(_NW, nchunks, _CHUNK), col3],
                 axis=2).reshape(_NW, 2 * nchunks, _CHUNK)

  halves = jnp.full((n, _DEGW), 0.5, jnp.float32)
  ones = jnp.ones((_CHUNK, _DEGW), jnp.float32)
  deg_pair = _make_deg(n, d, e)(halves, ones, col3)

  agg = _make_agg(n, d, e)
  r = 1000

  u1, disb = _tc_call(_first_body, n, d, r, (x, deg_pair[0], deg_pair[1]),
                      (), (W1,), num_out=2, num_narrow=2)
  a1 = agg(u1, rc)
  u2 = _tc_call(_mid_body, n, d, r, (a1[0], a1[1], u1, disb),
                (b1.reshape(1, d), g1.reshape(1, d), be1.reshape(1, d)), (W2,))
  a2 = agg(u2, rc)
  u3 = _tc_call(_mid_body, n, d, r, (a2[0], a2[1], u2, disb),
                (b2.reshape(1, d), g2.reshape(1, d), be2.reshape(1, d)), (W3,))
  a3 = agg(u3, rc)
  out = _tc_call(_last_body, n, d, r, (a3[0], a3[1], u3, disb),
                 (b3.reshape(1, d), g3.reshape(1, d), be3.reshape(1, d)), ())
  return out


# speedup vs baseline: 1.1717x; 1.1717x over previous
import functools
import math

import jax
import jax.numpy as jnp
from jax import lax
from jax.experimental import pallas as pl
from jax.experimental.pallas import tpu as pltpu
from jax.experimental.pallas import tpu_sc as plsc

_NC = 2
_NS = 16
_NW = _NC * _NS
_CHUNK = 88
_BN_C = 1.0 / math.sqrt(1.0 + 1e-5)


def _row_init(sid, src, dst, rows_pt, rows_tail):
  base_r = sid * rows_pt
  pltpu.sync_copy(src.at[pl.ds(base_r, rows_pt)], dst.at[pl.ds(base_r, rows_pt)])
  if rows_tail:
    @pl.when(sid == _NS - 1)
    def _tail():
      pltpu.sync_copy(src.at[pl.ds(rows_pt * _NS, rows_tail)],
                      dst.at[pl.ds(rows_pt * _NS, rows_tail)])


_PADR = 16


def _splits(n, d, e):
  epw = e // _NW
  nchunks = -(-epw // _CHUNK)
  assert epw * _NW == e and d % 128 == 0
  rows_pt = (n // _NS) & ~7
  rows_tail = n - rows_pt * _NS
  assert rows_tail >= 0 and rows_tail % 8 == 0
  return nchunks, rows_pt, rows_tail


@functools.lru_cache(maxsize=None)
def _make_agg(n, d, e):
  nchunks, rows_pt, rows_tail = _splits(n, d, e)
  nmain = (nchunks // 8) * 8
  tail = list(range(nmain, nchunks))
  assert len(tail) <= 6
  mesh = plsc.VectorSubcoreMesh(core_axis_name="c", subcore_axis_name="s")

  @functools.partial(
      pl.kernel,
      out_type=jax.ShapeDtypeStruct((_NC, n, d), jnp.float32),
      mesh=mesh,
      scratch_types=[
          pltpu.VMEM((16, _CHUNK), jnp.int32),
          pltpu.VMEM((4, _CHUNK, d), jnp.float32),
          pltpu.VMEM_SHARED((n + _PADR, d), jnp.float32),
      ] + [pltpu.SemaphoreType.DMA] * 16,
  )
  def agg(u_hbm, rc_hbm, out_hbm, rcring, gbuf, acc, *sems):
    gsem = sems[0:4]
    ssem = sems[4:8]
    isem = sems[8:16]
    cid = lax.axis_index("c")
    sid = lax.axis_index("s")
    wid = sid * _NC + cid
    _row_init(sid, u_hbm, acc, rows_pt, rows_tail)

    def start_fetch(ci, i8):
      pltpu.async_copy(rc_hbm.at[wid].at[pl.ds(2 * ci, 2)],
                       rcring.at[pl.ds(2 * i8, 2)], isem[i8])

    def wait_fetch(i8):
      pltpu.make_async_copy(rc_hbm.at[0].at[pl.ds(0, 2)],
                            rcring.at[pl.ds(2 * i8, 2)], isem[i8]).wait()

    def start_gather(i8, g):
      pltpu.async_copy(u_hbm.at[rcring.at[2 * i8]], gbuf.at[g], gsem[g])

    def wait_gather(g):
      pltpu.make_async_copy(u_hbm.at[rcring.at[0]], gbuf.at[g],
                            gsem[g]).wait()

    def start_scatter(i8, g):
      pltpu.async_copy(gbuf.at[g], acc.at[rcring.at[2 * i8 + 1]], ssem[g],
                       add=True)

    def wait_scatter(g):
      pltpu.make_async_copy(gbuf.at[g], acc.at[rcring.at[0]], ssem[g]).wait()

    for c in range(4):
      start_fetch(c, c)
    plsc.subcore_barrier()

    def body(k, carry):
      c0 = 8 * k
      for b in range(8):
        ci = c0 + b

        @pl.when(ci >= 4)
        def _recycle():
          wait_scatter(b % 4)

        @pl.when(ci + 4 < nmain)
        def _pref():
          start_fetch(ci + 4, (b + 4) % 8)

        wait_fetch(b)
        start_gather(b, b % 4)

        if b >= 2:
          wait_gather((b - 2) % 4)
          start_scatter((b - 2) % 8, (b - 2) % 4)
        else:
          @pl.when(ci >= 2)
          def _scat():
            wait_gather((b - 2) % 4)
            start_scatter((b - 2) % 8, (b - 2) % 4)

      return carry

    lax.fori_loop(0, nmain // 8, body, 0)
    for ci in tail:
      wait_scatter(ci % 4)
      start_fetch(ci, ci % 8)
      wait_fetch(ci % 8)
      start_gather(ci % 8, ci % 4)
      wait_gather((ci - 2) % 4)
      start_scatter((ci - 2) % 8, (ci - 2) % 4)
    for ci in (nchunks - 2, nchunks - 1):
      wait_gather(ci % 4)
      start_scatter(ci % 8, ci % 4)
    for ci in range(nchunks - 4, nchunks):
      wait_scatter(ci % 4)
    plsc.subcore_barrier()
    _row_init(sid, acc, out_hbm.at[cid], rows_pt, rows_tail)

  return agg


_DEGW = 64


@functools.lru_cache(maxsize=None)
def _make_deg(n, d, e):
  nchunks, rows_pt, rows_tail = _splits(n, d, e)
  window = 8
  mesh = plsc.VectorSubcoreMesh(core_axis_name="c", subcore_axis_name="s")

  @functools.partial(
      pl.kernel,
      out_type=jax.ShapeDtypeStruct((_NC, n, _DEGW), jnp.float32),
      mesh=mesh,
      scratch_types=[
          pltpu.VMEM((nchunks, _CHUNK), jnp.int32),
          pltpu.VMEM((_CHUNK, _DEGW), jnp.float32),
          pltpu.VMEM_SHARED((n + _PADR, _DEGW), jnp.float32),
          pltpu.SemaphoreType.DMA,
      ],
  )
  def deg(half_hbm, ones_hbm, col3_hbm, out_hbm, col_v, ones_v, acc, ssem):
    cid = lax.axis_index("c")
    sid = lax.axis_index("s")
    wid = sid * _NC + cid
    pltpu.sync_copy(col3_hbm.at[wid], col_v)
    pltpu.sync_copy(ones_hbm.at[pl.ds(0, _CHUNK)], ones_v)
    _row_init(sid, half_hbm, acc, rows_pt, rows_tail)
    plsc.subcore_barrier()

    def start_sc(ci):
      pltpu.async_copy(ones_v, acc.at[col_v.at[ci]], ssem, add=True)

    def wait_sc(i, carry):
      pltpu.make_async_copy(ones_v, acc.at[col_v.at[0]], ssem).wait()
      return carry

    for ci in range(window):
      start_sc(ci)

    def body(k, carry):
      wait_sc(k, carry)
      start_sc(k + window)
      return carry

    lax.fori_loop(0, nchunks - window, body, 0)
    lax.fori_loop(0, window, wait_sc, 0)
    plsc.subcore_barrier()
    _row_init(sid, acc, out_hbm.at[cid], rows_pt, rows_tail)

  return deg


def _first_body(x, d0, d1, w, o_u, o_dis):
  deg = d0[...][:, :1] + d1[...][:, :1]
  dis = jnp.broadcast_to(lax.rsqrt(deg), o_dis.shape)
  o_dis[...] = dis
  o_u[...] = jnp.dot(x[...], w[...], preferred_element_type=jnp.float32) * dis


def _mid_body(a0, a1, up, dis, b, g, be, w, o):
  d = dis[...]
  z = (a0[...] + a1[...] - up[...]) * d + b[...]
  z = z * (g[...] * _BN_C) + be[...]
  z = jnp.maximum(z, 0.0)
  o[...] = jnp.dot(z, w[...], preferred_element_type=jnp.float32) * d


def _last_body(a0, a1, up, dis, b, g, be, o):
  z = (a0[...] + a1[...] - up[...]) * dis[...] + b[...]
  o[...] = z * (g[...] * _BN_C) + be[...]


def _tc_call(body, n, d, r, arrs, vecs, weights, num_out=1, num_narrow=0):
  grid = (n // r,)
  nd_spec = pl.BlockSpec((r, d), lambda i: (i, 0))
  nr_spec = pl.BlockSpec((r, _DEGW), lambda i: (i, 0))
  vec_spec = pl.BlockSpec((1, d), lambda i: (0, 0))
  w_spec = pl.BlockSpec((d, d), lambda i: (0, 0))
  in_specs = ([nd_spec] * (len(arrs) - num_narrow) + [nr_spec] * num_narrow
              + [vec_spec] * len(vecs) + [w_spec] * len(weights))
  shape = jax.ShapeDtypeStruct((n, d), jnp.float32)
  return pl.pallas_call(
      body,
      grid=grid,
      in_specs=in_specs,
      out_specs=[nd_spec] * num_out if num_out > 1 else nd_spec,
      out_shape=[shape] * num_out if num_out > 1 else shape,
  )(*arrs, *vecs, *weights)


def kernel(x, edge_index, W1, b1, g1, be1, W2, b2, g2, be2, W3, b3, g3, be3):
  n, d = x.shape
  e = edge_index.shape[1]
  epw = e // _NW
  nchunks = -(-epw // _CHUNK)
  pad = nchunks * _CHUNK - epw

  w_ids = jnp.arange(_NW, dtype=jnp.int32)[:, None]
  p_ids = jnp.arange(pad, dtype=jnp.int32)[None, :]
  rowp = jnp.concatenate(
      [edge_index[0].reshape(_NW, epw), (p_ids + 113 * w_ids) % n], axis=1)
  colp = jnp.concatenate(
      [edge_index[1].reshape(_NW, epw), n + (p_ids + w_ids) % _PADR], axis=1)
  col3 = colp.reshape(_NW, nchunks, _CHUNK)
  rc = jnp.stack([rowp.reshape(_NW, nchunks, _CHUNK), col3],
                 axis=2).reshape(_NW, 2 * nchunks, _CHUNK)

  halves = jnp.full((n, _DEGW), 0.5, jnp.float32)
  ones = jnp.ones((_CHUNK, _DEGW), jnp.float32)
  deg_pair = _make_deg(n, d, e)(halves, ones, col3)

  agg = _make_agg(n, d, e)
  r = 1000

  u1, disb = _tc_call(_first_body, n, d, r, (x, deg_pair[0], deg_pair[1]),
                      (), (W1,), num_out=2, num_narrow=2)
  a1 = agg(u1, rc)
  u2 = _tc_call(_mid_body, n, d, r, (a1[0], a1[1], u1, disb),
                (b1.reshape(1, d), g1.reshape(1, d), be1.reshape(1, d)), (W2,))
  a2 = agg(u2, rc)
  u3 = _tc_call(_mid_body, n, d, r, (a2[0], a2[1], u2, disb),
                (b2.reshape(1, d), g2.reshape(1, d), be2.reshape(1, d)), (W3,))
  a3 = agg(u3, rc)
  out = _tc_call(_last_body, n, d, r, (a3[0], a3[1], u3, disb),
                 (b3.reshape(1, d), g3.reshape(1, d), be3.reshape(1, d)), ())
  return out
